# baseline (speedup 1.0000x reference)
import functools

import jax
import jax.numpy as jnp
from jax import lax
from jax.experimental import pallas as pl
from jax.experimental.pallas import tpu as pltpu
from jax.experimental.pallas import tpu_sc as plsc

N = 10000
E = 320000
D = 128

NC = 2
NS = 16
NW = NC * NS
HALVES = 2
EC = E // HALVES
PER_TILE = EC // NW
CS = 40
NCH = PER_TILE // CS
GK = 5
NG = NCH // GK
N_PAD = 10240
NROWS_T = N_PAD // NS

_mesh = plsc.VectorSubcoreMesh(
    core_axis_name="c", subcore_axis_name="s", num_cores=NC, num_subcores=NS)


def _pipeline(ng, issue_a, drain_a, issue_b, drain_b):
    issue_a(0, 0)

    def body(gg, carry):
        for p in (0, 1):
            g = 2 * gg + p

            @pl.when(g >= 2)
            def _():
                drain_b(g - 2, p)

            @pl.when(g >= 1)
            def _():
                issue_a(g, p)

            @pl.when(g >= 1)
            def _():
                drain_a(g - 1, 1 - p)
                issue_b(g - 1, 1 - p)

        return carry

    lax.fori_loop(0, ng // 2, body, 0)
    if ng % 2:
        drain_b(ng - 3, 0)
        issue_a(ng - 1, 0)
        drain_a(ng - 2, 1)
        issue_b(ng - 2, 1)
        drain_a(ng - 1, 0)
        issue_b(ng - 1, 0)
        drain_b(ng - 2, 1)
        drain_b(ng - 1, 0)
    else:
        drain_a(ng - 1, 1)
        issue_b(ng - 1, 1)
        drain_b(ng - 2, 0)
        drain_b(ng - 1, 1)


def _pack(xf, af):
    xu = lax.bitcast_convert_type(xf, jnp.uint32) >> 16
    au = lax.bitcast_convert_type(af, jnp.uint32) & jnp.uint32(0xFFFF0000)
    return lax.bitcast_convert_type(xu | au, jnp.int32)


def _unpack(w):
    wu = lax.bitcast_convert_type(w, jnp.uint32)
    lo = lax.bitcast_convert_type(wu << 16, jnp.float32)
    hi = lax.bitcast_convert_type(wu & jnp.uint32(0xFFFF0000), jnp.float32)
    return lo, hi


BP = 2000


def _pre_body(xb, wpe, wce, m1p, m1c, bpe, bce, ts, td):
    xv = xb[...]
    t1 = jnp.dot(xv, wpe[...], preferred_element_type=jnp.float32) + bpe[...]
    t1 = jnp.where(t1 >= 0, t1, 0.01 * t1)
    a = jnp.dot(t1, m1p[...], preferred_element_type=jnp.float32)
    t2 = jnp.dot(xv, wce[...], preferred_element_type=jnp.float32) + bce[...]
    t2 = jnp.where(t2 >= 0, t2, 0.01 * t2)
    b = jnp.dot(t2, m1c[...], preferred_element_type=jnp.float32)
    xr = xv.astype(jnp.bfloat16).astype(jnp.float32)
    ts[...] = _pack(xr, a.astype(jnp.bfloat16).astype(jnp.float32))
    td[...] = _pack(xr, b.astype(jnp.bfloat16).astype(jnp.float32))


def _pre(x, wpe, wce, m1p, m1c, bpe, bce):
    full = lambda shp: pl.BlockSpec(shp, lambda i: (0,) * len(shp))
    return pl.pallas_call(
        _pre_body,
        grid=(N // BP,),
        in_specs=[
            pl.BlockSpec((BP, D), lambda i: (i, 0)),
            full((D, D)), full((D, D)), full((D, D)), full((D, D)),
            full((1, D)), full((1, D)),
        ],
        out_specs=[pl.BlockSpec((BP, D), lambda i: (i, 0)),
                   pl.BlockSpec((BP, D), lambda i: (i, 0))],
        out_shape=[jax.ShapeDtypeStruct((N, D), jnp.int32),
                   jax.ShapeDtypeStruct((N, D), jnp.int32)],
    )(x, wpe, wce, m1p, m1c, bpe, bce)


@functools.partial(
    pl.kernel,
    out_type=[jax.ShapeDtypeStruct((EC, D), jnp.int32),
              jax.ShapeDtypeStruct((EC, D), jnp.int32)],
    mesh=_mesh,
    scratch_types=[
        pltpu.VMEM((PER_TILE,), jnp.int32),
        pltpu.VMEM((PER_TILE,), jnp.int32),
        pltpu.VMEM((2 * GK, CS, D), jnp.int32),
        pltpu.VMEM((2 * GK, CS, D), jnp.int32),
    ] + [pltpu.SemaphoreType.DMA] * 8,
)
def _sc_gather(ts_hbm, td_hbm, src_hbm, dst_hbm, gs_hbm, gd_hbm,
               idx_s, idx_d, buf_s, buf_d,
               sg_s0, sg_s1, sg_d0, sg_d1, sw_s0, sw_s1, sw_d0, sw_d1):
    cid = lax.axis_index("c")
    sid = lax.axis_index("s")
    wid = cid * NS + sid
    base = wid * PER_TILE
    pltpu.sync_copy(src_hbm.at[pl.ds(base, PER_TILE)], idx_s)
    pltpu.sync_copy(dst_hbm.at[pl.ds(base, PER_TILE)], idx_d)
    sg = ((sg_s0, sg_d0), (sg_s1, sg_d1))
    sw = ((sw_s0, sw_d0), (sw_s1, sw_d1))

    def issue_gathers(g, p):
        for i in range(GK):
            off = (g * GK + i) * CS
            k = p * GK + i
            pltpu.async_copy(ts_hbm.at[idx_s.at[pl.ds(off, CS)]],
                             buf_s.at[k], sg[p][0])
            pltpu.async_copy(td_hbm.at[idx_d.at[pl.ds(off, CS)]],
                             buf_d.at[k], sg[p][1])

    def drain_gathers(g, p):
        for i in range(GK):
            off = (g * GK + i) * CS
            k = p * GK + i
            pltpu.make_async_copy(ts_hbm.at[idx_s.at[pl.ds(off, CS)]],
                                  buf_s.at[k], sg[p][0]).wait()
            pltpu.make_async_copy(td_hbm.at[idx_d.at[pl.ds(off, CS)]],
                                  buf_d.at[k], sg[p][1]).wait()

    def issue_writes(g, p):
        for i in range(GK):
            off = (g * GK + i) * CS
            k = p * GK + i
            pltpu.async_copy(buf_s.at[k], gs_hbm.at[pl.ds(base + off, CS)],
                             sw[p][0])
            pltpu.async_copy(buf_d.at[k], gd_hbm.at[pl.ds(base + off, CS)],
                             sw[p][1])

    def drain_writes(g, p):
        for i in range(GK):
            off = (g * GK + i) * CS
            k = p * GK + i
            pltpu.make_async_copy(buf_s.at[k],
                                  gs_hbm.at[pl.ds(base + off, CS)],
                                  sw[p][0]).wait()
            pltpu.make_async_copy(buf_d.at[k],
                                  gd_hbm.at[pl.ds(base + off, CS)],
                                  sw[p][1]).wait()

    _pipeline(NG, issue_gathers, drain_gathers, issue_writes, drain_writes)


CS2 = 125
NCH2 = PER_TILE // CS2
NG2 = NCH2
WB = 160


@functools.partial(
    pl.kernel,
    out_type=jax.ShapeDtypeStruct((NC, N_PAD, D), jnp.float32),
    mesh=_mesh,
    scratch_types=[
        pltpu.VMEM((NCH2, CS2), jnp.int32),
        pltpu.VMEM((256, D), jnp.float32),
        pltpu.VMEM_SHARED((N_PAD, D), jnp.float32),
    ] + [pltpu.SemaphoreType.DMA] * 4,
)
def _sc_scatter(msg3_hbm, dst3_hbm, out_hbm, idx_all, rows, agg_sh,
                sr0, sr1, sa0, sa1):
    cid = lax.axis_index("c")
    sid = lax.axis_index("s")
    wid = cid * NS + sid

    def zb(t, carry):
        i = t // (D // 16)
        k = t % (D // 16)
        rows[i, pl.ds(k * 16, 16)] = jnp.zeros((16,), jnp.float32)
        return carry

    lax.fori_loop(0, WB * (D // 16), zb, 0)
    row0 = sid * NROWS_T
    for m in range(NROWS_T // WB):
        pltpu.sync_copy(rows.at[pl.ds(0, WB)],
                        agg_sh.at[pl.ds(row0 + m * WB, WB)])
    plsc.subcore_barrier()

    cbase = wid * NCH2
    pltpu.sync_copy(dst3_hbm.at[wid], idx_all)
    sr = (sr0, sr1)
    sa = (sa0, sa1)

    def issue_reads(g, p):
        pltpu.async_copy(msg3_hbm.at[cbase + g],
                         rows.at[pl.ds(p * 128, CS2)], sr[p])

    def drain_reads(g, p):
        pltpu.make_async_copy(msg3_hbm.at[cbase + g],
                              rows.at[pl.ds(p * 128, CS2)], sr[p]).wait()

    def issue_adds(g, p):
        pltpu.async_copy(rows.at[pl.ds(p * 128, CS2)],
                         agg_sh.at[idx_all.at[g]], sa[p], add=True)

    def drain_adds(g, p):
        pltpu.make_async_copy(rows.at[pl.ds(p * 128, CS2)],
                              agg_sh.at[idx_all.at[g]], sa[p]).wait()

    _pipeline(NG2, issue_reads, drain_reads, issue_adds, drain_adds)
    plsc.subcore_barrier()

    for m in range(NROWS_T // WB):
        r = row0 + m * WB
        pltpu.sync_copy(agg_sh.at[pl.ds(r, WB)], rows.at[pl.ds(0, WB)])
        pltpu.sync_copy(rows.at[pl.ds(0, WB)],
                        out_hbm.at[cid].at[pl.ds(r, WB)])


BE = 3200


def _edge_body(gs, gd, m1d, bm1, w2, bm2, msg):
    xs, a_s = _unpack(gs[...])
    xd, b_d = _unpack(gd[...])
    diff = jnp.abs(xs - xd).astype(jnp.bfloat16)
    pre = (jnp.dot(diff, m1d[...], preferred_element_type=jnp.float32)
           + a_s + b_d + bm1[...])
    h = jnp.maximum(pre, 0.0)
    z = jnp.sum(h * w2[...], axis=1, keepdims=True) + bm2[...]
    ew = 1.0 / (1.0 + jnp.exp(-z))
    msg[...] = xs * ew


def _edge_mlp(gs, gd, m1d, bm1, w2, bm2):
    full = lambda shp: pl.BlockSpec(shp, lambda i: (0,) * len(shp))
    return pl.pallas_call(
        _edge_body,
        grid=(EC // BE,),
        in_specs=[
            pl.BlockSpec((BE, D), lambda i: (i, 0)),
            pl.BlockSpec((BE, D), lambda i: (i, 0)),
            full((D, D)),
            full((1, D)), full((1, D)), full((1, 1)),
        ],
        out_specs=pl.BlockSpec((BE, D), lambda i: (i, 0)),
        out_shape=jax.ShapeDtypeStruct((EC, D), jnp.float32),
    )(gs, gd, m1d, bm1, w2, bm2)


BN = 2000


def _post_body(a0, a1, a2, a3, xb, wrel, wroot, brel, out):
    agg = (a0[...] + a1[...]) + (a2[...] + a3[...])
    out[...] = (jnp.dot(agg, wrel[...], preferred_element_type=jnp.float32)
                + jnp.dot(xb[...], wroot[...],
                          preferred_element_type=jnp.float32)
                + brel[...])


def _post(a0, a1, a2, a3, x, wrel, wroot, brel):
    full = lambda shp: pl.BlockSpec(shp, lambda i: (0,) * len(shp))
    blk = pl.BlockSpec((BN, D), lambda i: (i, 0))
    return pl.pallas_call(
        _post_body,
        grid=(N // BN,),
        in_specs=[blk, blk, blk, blk, blk,
                  full((D, D)), full((D, D)), full((1, D))],
        out_specs=blk,
        out_shape=jax.ShapeDtypeStruct((N, D), jnp.float32),
    )(a0, a1, a2, a3, x, wrel, wroot, brel)


def kernel(x, edge_index, W_pe, b_pe, W_ce, b_ce, W_m1, b_m1, W_m2, b_m2,
           W_rel, b_rel, W_root):
    src = edge_index[0]
    dst = edge_index[1]
    dst4 = dst.reshape(HALVES, NW, NCH2, CS2)
    bf = jnp.bfloat16

    ts, td = _pre(x, W_pe, W_ce, W_m1[:D], W_m1[D:2 * D],
                  b_pe.reshape(1, D), b_ce.reshape(1, D))

    m1d = W_m1[2 * D:].astype(bf)
    bm1 = b_m1.reshape(1, D)
    w2 = W_m2.reshape(1, D)
    bm2 = b_m2.reshape(1, 1)

    parts = []
    for c in range(HALVES):
        gs, gd = _sc_gather(ts, td, src[c * EC:(c + 1) * EC],
                            dst[c * EC:(c + 1) * EC])
        msg = _edge_mlp(gs, gd, m1d, bm1, w2, bm2)
        parts.append(_sc_scatter(msg.reshape(NW * NCH2, CS2, D), dst4[c]))

    return _post(parts[0][0], parts[0][1], parts[1][0], parts[1][1],
                 x, W_rel, W_root, b_rel.reshape(1, D))


# speedup vs baseline: 1.1622x; 1.1622x over previous
import functools

import jax
import jax.numpy as jnp
from jax import lax
from jax.experimental import pallas as pl
from jax.experimental.pallas import tpu as pltpu
from jax.experimental.pallas import tpu_sc as plsc

N = 10000
E = 320000
D = 128

NC = 2
NS = 16
NW = NC * NS
HALVES = 2
EC = E // HALVES
PER_TILE = EC // NW
CS = 40
NCH = PER_TILE // CS
GK = 5
NG = NCH // GK
N_PAD = 10240
NROWS_T = N_PAD // NS

_mesh = plsc.VectorSubcoreMesh(
    core_axis_name="c", subcore_axis_name="s", num_cores=NC, num_subcores=NS)


def _pipeline(ng, issue_a, drain_a, issue_b, drain_b):
    issue_a(0, 0)

    def body(gg, carry):
        for p in (0, 1):
            g = 2 * gg + p

            @pl.when(g >= 2)
            def _():
                drain_b(g - 2, p)

            @pl.when(g >= 1)
            def _():
                issue_a(g, p)

            @pl.when(g >= 1)
            def _():
                drain_a(g - 1, 1 - p)
                issue_b(g - 1, 1 - p)

        return carry

    lax.fori_loop(0, ng // 2, body, 0)
    if ng % 2:
        drain_b(ng - 3, 0)
        issue_a(ng - 1, 0)
        drain_a(ng - 2, 1)
        issue_b(ng - 2, 1)
        drain_a(ng - 1, 0)
        issue_b(ng - 1, 0)
        drain_b(ng - 2, 1)
        drain_b(ng - 1, 0)
    else:
        drain_a(ng - 1, 1)
        issue_b(ng - 1, 1)
        drain_b(ng - 2, 0)
        drain_b(ng - 1, 1)


def _pack(xf, af):
    xu = lax.bitcast_convert_type(xf, jnp.uint32) >> 16
    au = lax.bitcast_convert_type(af, jnp.uint32) & jnp.uint32(0xFFFF0000)
    return lax.bitcast_convert_type(xu | au, jnp.int32)


def _unpack(w):
    wu = lax.bitcast_convert_type(w, jnp.uint32)
    lo = lax.bitcast_convert_type(wu << 16, jnp.float32)
    hi = lax.bitcast_convert_type(wu & jnp.uint32(0xFFFF0000), jnp.float32)
    return lo, hi


BP = 2000


def _pre_body(xb, wpe, wce, m1p, m1c, bpe, bce, ts, td):
    xv = xb[...]
    t1 = jnp.dot(xv, wpe[...], preferred_element_type=jnp.float32) + bpe[...]
    t1 = jnp.where(t1 >= 0, t1, 0.01 * t1)
    a = jnp.dot(t1, m1p[...], preferred_element_type=jnp.float32)
    t2 = jnp.dot(xv, wce[...], preferred_element_type=jnp.float32) + bce[...]
    t2 = jnp.where(t2 >= 0, t2, 0.01 * t2)
    b = jnp.dot(t2, m1c[...], preferred_element_type=jnp.float32)
    xr = xv.astype(jnp.bfloat16).astype(jnp.float32)
    ts[...] = _pack(xr, a.astype(jnp.bfloat16).astype(jnp.float32))
    td[...] = _pack(xr, b.astype(jnp.bfloat16).astype(jnp.float32))


def _pre(x, wpe, wce, m1p, m1c, bpe, bce):
    full = lambda shp: pl.BlockSpec(shp, lambda i: (0,) * len(shp))
    return pl.pallas_call(
        _pre_body,
        grid=(N // BP,),
        in_specs=[
            pl.BlockSpec((BP, D), lambda i: (i, 0)),
            full((D, D)), full((D, D)), full((D, D)), full((D, D)),
            full((1, D)), full((1, D)),
        ],
        out_specs=[pl.BlockSpec((BP, D), lambda i: (i, 0)),
                   pl.BlockSpec((BP, D), lambda i: (i, 0))],
        out_shape=[jax.ShapeDtypeStruct((N, D), jnp.int32),
                   jax.ShapeDtypeStruct((N, D), jnp.int32)],
    )(x, wpe, wce, m1p, m1c, bpe, bce)


@functools.partial(
    pl.kernel,
    out_type=[jax.ShapeDtypeStruct((EC, D), jnp.int32),
              jax.ShapeDtypeStruct((EC, D), jnp.int32)],
    mesh=_mesh,
    scratch_types=[
        pltpu.VMEM((PER_TILE,), jnp.int32),
        pltpu.VMEM((PER_TILE,), jnp.int32),
        pltpu.VMEM((2 * GK, CS, D), jnp.int32),
        pltpu.VMEM((2 * GK, CS, D), jnp.int32),
    ] + [pltpu.SemaphoreType.DMA] * 8,
)
def _sc_gather(ts_hbm, td_hbm, src_hbm, dst_hbm, gs_hbm, gd_hbm,
               idx_s, idx_d, buf_s, buf_d,
               sg_s0, sg_s1, sg_d0, sg_d1, sw_s0, sw_s1, sw_d0, sw_d1):
    cid = lax.axis_index("c")
    sid = lax.axis_index("s")
    wid = cid * NS + sid
    base = wid * PER_TILE
    pltpu.sync_copy(src_hbm.at[pl.ds(base, PER_TILE)], idx_s)
    pltpu.sync_copy(dst_hbm.at[pl.ds(base, PER_TILE)], idx_d)
    sg = ((sg_s0, sg_d0), (sg_s1, sg_d1))
    sw = ((sw_s0, sw_d0), (sw_s1, sw_d1))

    def issue_gathers(g, p):
        for i in range(GK):
            off = (g * GK + i) * CS
            k = p * GK + i
            pltpu.async_copy(ts_hbm.at[idx_s.at[pl.ds(off, CS)]],
                             buf_s.at[k], sg[p][0])
            pltpu.async_copy(td_hbm.at[idx_d.at[pl.ds(off, CS)]],
                             buf_d.at[k], sg[p][1])

    def drain_gathers(g, p):
        for i in range(GK):
            off = (g * GK + i) * CS
            k = p * GK + i
            pltpu.make_async_copy(ts_hbm.at[idx_s.at[pl.ds(off, CS)]],
                                  buf_s.at[k], sg[p][0]).wait()
            pltpu.make_async_copy(td_hbm.at[idx_d.at[pl.ds(off, CS)]],
                                  buf_d.at[k], sg[p][1]).wait()

    def issue_writes(g, p):
        for i in range(GK):
            off = (g * GK + i) * CS
            k = p * GK + i
            pltpu.async_copy(buf_s.at[k], gs_hbm.at[pl.ds(base + off, CS)],
                             sw[p][0])
            pltpu.async_copy(buf_d.at[k], gd_hbm.at[pl.ds(base + off, CS)],
                             sw[p][1])

    def drain_writes(g, p):
        for i in range(GK):
            off = (g * GK + i) * CS
            k = p * GK + i
            pltpu.make_async_copy(buf_s.at[k],
                                  gs_hbm.at[pl.ds(base + off, CS)],
                                  sw[p][0]).wait()
            pltpu.make_async_copy(buf_d.at[k],
                                  gd_hbm.at[pl.ds(base + off, CS)],
                                  sw[p][1]).wait()

    _pipeline(NG, issue_gathers, drain_gathers, issue_writes, drain_writes)


CS2 = 40
NCH2 = PER_TILE // CS2
NG2 = NCH2
WB = 160


@functools.partial(
    pl.kernel,
    out_type=jax.ShapeDtypeStruct((NC, N_PAD, D), jnp.float32),
    mesh=_mesh,
    scratch_types=[
        pltpu.VMEM((NCH2, CS2), jnp.int32),
        pltpu.VMEM((256, D), jnp.float32),
        pltpu.VMEM_SHARED((N_PAD, D), jnp.float32),
    ] + [pltpu.SemaphoreType.DMA] * 4,
)
def _sc_scatter(msg3_hbm, dst3_hbm, out_hbm, idx_all, rows, agg_sh,
                sr0, sr1, sa0, sa1):
    cid = lax.axis_index("c")
    sid = lax.axis_index("s")
    wid = cid * NS + sid

    def zb(t, carry):
        i = t // (D // 16)
        k = t % (D // 16)
        rows[i, pl.ds(k * 16, 16)] = jnp.zeros((16,), jnp.float32)
        return carry

    lax.fori_loop(0, WB * (D // 16), zb, 0)
    row0 = sid * NROWS_T
    for m in range(NROWS_T // WB):
        pltpu.sync_copy(rows.at[pl.ds(0, WB)],
                        agg_sh.at[pl.ds(row0 + m * WB, WB)])
    plsc.subcore_barrier()

    cbase = wid * NCH2
    pltpu.sync_copy(dst3_hbm.at[wid], idx_all)
    sr = (sr0, sr1)
    sa = (sa0, sa1)

    def issue_reads(g, p):
        pltpu.async_copy(msg3_hbm.at[cbase + g],
                         rows.at[pl.ds(p * 128, CS2)], sr[p])

    def drain_reads(g, p):
        pltpu.make_async_copy(msg3_hbm.at[cbase + g],
                              rows.at[pl.ds(p * 128, CS2)], sr[p]).wait()

    def issue_adds(g, p):
        pltpu.async_copy(rows.at[pl.ds(p * 128, CS2)],
                         agg_sh.at[idx_all.at[g]], sa[p], add=True)

    def drain_adds(g, p):
        pltpu.make_async_copy(rows.at[pl.ds(p * 128, CS2)],
                              agg_sh.at[idx_all.at[g]], sa[p]).wait()

    _pipeline(NG2, issue_reads, drain_reads, issue_adds, drain_adds)
    plsc.subcore_barrier()

    for m in range(NROWS_T // WB):
        r = row0 + m * WB
        pltpu.sync_copy(agg_sh.at[pl.ds(r, WB)], rows.at[pl.ds(0, WB)])
        pltpu.sync_copy(rows.at[pl.ds(0, WB)],
                        out_hbm.at[cid].at[pl.ds(r, WB)])


BE = 3200


def _edge_body(gs, gd, m1d, bm1, w2, bm2, msg):
    xs, a_s = _unpack(gs[...])
    xd, b_d = _unpack(gd[...])
    diff = jnp.abs(xs - xd).astype(jnp.bfloat16)
    pre = (jnp.dot(diff, m1d[...], preferred_element_type=jnp.float32)
           + a_s + b_d + bm1[...])
    h = jnp.maximum(pre, 0.0)
    z = jnp.sum(h * w2[...], axis=1, keepdims=True) + bm2[...]
    ew = 1.0 / (1.0 + jnp.exp(-z))
    msg[...] = xs * ew


def _edge_mlp(gs, gd, m1d, bm1, w2, bm2):
    full = lambda shp: pl.BlockSpec(shp, lambda i: (0,) * len(shp))
    return pl.pallas_call(
        _edge_body,
        grid=(EC // BE,),
        in_specs=[
            pl.BlockSpec((BE, D), lambda i: (i, 0)),
            pl.BlockSpec((BE, D), lambda i: (i, 0)),
            full((D, D)),
            full((1, D)), full((1, D)), full((1, 1)),
        ],
        out_specs=pl.BlockSpec((BE, D), lambda i: (i, 0)),
        out_shape=jax.ShapeDtypeStruct((EC, D), jnp.float32),
    )(gs, gd, m1d, bm1, w2, bm2)


BN = 2000


def _post_body(a0, a1, a2, a3, xb, wrel, wroot, brel, out):
    agg = (a0[...] + a1[...]) + (a2[...] + a3[...])
    out[...] = (jnp.dot(agg, wrel[...], preferred_element_type=jnp.float32)
                + jnp.dot(xb[...], wroot[...],
                          preferred_element_type=jnp.float32)
                + brel[...])


def _post(a0, a1, a2, a3, x, wrel, wroot, brel):
    full = lambda shp: pl.BlockSpec(shp, lambda i: (0,) * len(shp))
    blk = pl.BlockSpec((BN, D), lambda i: (i, 0))
    return pl.pallas_call(
        _post_body,
        grid=(N // BN,),
        in_specs=[blk, blk, blk, blk, blk,
                  full((D, D)), full((D, D)), full((1, D))],
        out_specs=blk,
        out_shape=jax.ShapeDtypeStruct((N, D), jnp.float32),
    )(a0, a1, a2, a3, x, wrel, wroot, brel)


def kernel(x, edge_index, W_pe, b_pe, W_ce, b_ce, W_m1, b_m1, W_m2, b_m2,
           W_rel, b_rel, W_root):
    src = edge_index[0]
    dst = edge_index[1]
    dst4 = dst.reshape(HALVES, NW, NCH2, CS2)
    bf = jnp.bfloat16

    ts, td = _pre(x, W_pe, W_ce, W_m1[:D], W_m1[D:2 * D],
                  b_pe.reshape(1, D), b_ce.reshape(1, D))

    m1d = W_m1[2 * D:].astype(bf)
    bm1 = b_m1.reshape(1, D)
    w2 = W_m2.reshape(1, D)
    bm2 = b_m2.reshape(1, 1)

    parts = []
    for c in range(HALVES):
        gs, gd = _sc_gather(ts, td, src[c * EC:(c + 1) * EC],
                            dst[c * EC:(c + 1) * EC])
        msg = _edge_mlp(gs, gd, m1d, bm1, w2, bm2)
        parts.append(_sc_scatter(msg.reshape(NW * NCH2, CS2, D), dst4[c]))

    return _post(parts[0][0], parts[0][1], parts[1][0], parts[1][1],
                 x, W_rel, W_root, b_rel.reshape(1, D))
